# Initial kernel scaffold; baseline (speedup 1.0000x reference)
import jax
import jax.numpy as jnp
from jax.experimental import pallas as pl


def kernel(x, edge_index, r, h, c, params):
    raise NotImplementedError("write your pallas kernel here")


# speedup vs baseline: 1.0004x; 1.0004x over previous
import jax
import jax.numpy as jnp
from jax.experimental import pallas as pl

B, T, N, FIN = 1, 40, 2912, 161
K_NN = 6


def _bn(xx, g, b):
    m = xx.mean(axis=(0, 2, 3), keepdims=True)
    v = xx.var(axis=(0, 2, 3), keepdims=True)
    return (xx - m) / jnp.sqrt(v + 1e-5) * g[None, :, None, None] + b[None, :, None, None]


def _gcn(xx, s, d, norm, theta):
    h = xx @ theta.T
    outs = []
    for t in range(h.shape[1]):
        ht = h[:, t]
        msg = jnp.take(ht, s, axis=1) * norm[None, :, None]
        outs.append(jnp.zeros_like(ht).at[:, d, :].add(msg))
    return jnp.stack(outs, axis=1)


def _antisym(xx, s, d, norm, W, b, theta):
    aw = W - W.T - 0.1 * jnp.eye(W.shape[0], dtype=xx.dtype)
    h = _gcn(xx, s, d, norm, theta)
    h = xx @ aw.T + h + b
    return xx + 0.1 * jnp.tanh(h)


def _outer_kernel(beta_ref, pr_ref, out_ref):
    out_ref[...] = beta_ref[...] * pr_ref[...]


def _pre_r_pallas(beta, pr):
    b2 = beta.reshape(1, N)
    p2 = pr.reshape(T, 1)
    out = pl.pallas_call(
        _outer_kernel,
        out_shape=jax.ShapeDtypeStruct((T, N), jnp.float32),
    )(b2, p2)
    return out[None]


def kernel(x, edge_index, r, h, c, params):
    p = params
    src, dst = edge_index[0], edge_index[1]
    loop = jnp.arange(N, dtype=src.dtype)
    s = jnp.concatenate([src, loop])
    d = jnp.concatenate([dst, loop])
    deg = jnp.zeros((N,), jnp.float32).at[d].add(1.0)
    dinv = jnp.where(deg > 0, 1.0 / jnp.sqrt(deg), 0.0)
    norm = dinv[s] * dinv[d]
    xt = jnp.transpose(x, (0, 2, 1, 3))
    xt = _bn(xt, p['bn1_g'], p['bn1_b'])
    x1 = jnp.transpose(xt, (0, 2, 1, 3))
    x1 = jnp.tanh(_antisym(x1, s, d, norm, p['as1_W'], p['as1_b'], p['as1_t']))
    x1 = jnp.tanh(_antisym(x1, s, d, norm, p['as2_W'], p['as2_b'], p['as2_t']))
    x1 = jnp.tanh(x1 @ p['fc0_W'].T + p['fc0_b'])
    x1 = jnp.transpose(x1, (0, 3, 2, 1))
    x1 = x1 @ p['l40_W'].T + p['l40_b']
    x1 = jnp.tanh(jnp.transpose(x1, (0, 3, 2, 1)))
    xs = x1[:, 0]
    ht = h[0]
    ct = c[0]
    for t in range(xs.shape[0]):
        g = xs[t] @ p['lstm_Wih'].T + p['lstm_bih'] + ht @ p['lstm_Whh'].T + p['lstm_bhh']
        i, f, gg, o = jnp.split(g, 4, axis=-1)
        ct = jax.nn.sigmoid(f) * ct + jax.nn.sigmoid(i) * jnp.tanh(gg)
        ht = jax.nn.sigmoid(o) * jnp.tanh(ct)
    H = ht
    mu = jnp.tanh(xs @ p['fc11_W'].T + p['fc11_b'])
    mu = jnp.tanh(mu @ p['fc12_W'].T + p['fc12_b'])
    mu = jnp.tanh(mu @ p['fc13_W'].T + p['fc13_b'])
    hw = xs @ p['fc2_W'].T
    temp = jnp.tanh(jnp.matmul(hw, jnp.transpose(hw, (0, 2, 1))))
    sq = jnp.sum(H * H, axis=1)
    d2 = sq[:, None] + sq[None, :] - 2.0 * (H @ H.T)
    D = jnp.sqrt(jnp.maximum(d2, 0.0))
    D = jax.lax.stop_gradient(D)
    _, idx = jax.lax.top_k(-D, K_NN + 1)
    idx = idx[:, 1:]
    rows = jnp.broadcast_to(jnp.arange(N)[:, None], idx.shape)
    Wm = jnp.zeros((N, N), jnp.int32).at[rows, idx].set(1)[None]
    xb = xs[:, None]
    beta = jnp.tanh(_bn(xb @ p['bl1_W'].T + p['bl1_b'], p['b1_g'], p['b1_b']))
    beta = jnp.tanh(_bn(beta @ p['bl2_W'].T + p['bl2_b'], p['b2_g'], p['b2_b']))
    beta = jnp.tanh(_bn(beta @ p['bl3_W'].T + p['bl3_b'], p['b3_g'], p['b3_b']))
    beta = (beta @ p['bl4_W'].T + p['bl4_b'])[:, 0]
    w = jnp.matmul(Wm.astype(jnp.float32), mu)
    wn = jax.nn.softmax(-50.0 * jnp.exp(-8.0 * w), axis=1) - jax.nn.softmax(-50.0 * jnp.exp(8.0 * w), axis=1)
    R = jnp.matmul(r, wn)[..., 0]
    pr = R[:, :, None]
    pre_r = _pre_r_pallas(beta, pr)
    return R, wn, xs, mu, temp, Wm, pre_r


# speedup vs baseline: 1.1049x; 1.1045x over previous
import functools

import jax
import jax.numpy as jnp
from jax.experimental import pallas as pl
from jax.experimental.pallas import tpu as pltpu

B, T, N, FIN = 1, 40, 2912, 161
K_NN = 6
F0 = 50
HID = 30


def _bf(a):
    return a.astype(jnp.bfloat16)


def _dotd(a, b, dims):
    return jax.lax.dot_general(_bf(a), _bf(b), (dims, ((), ())),
                               preferred_element_type=jnp.float32)


def _heads_body(xs_ref, f11_ref, f11b_ref, f12_ref, f12b_ref, f13_ref,
                fc2_ref, bl1_ref, bl1b_ref, bl2_ref, bl2b_ref,
                bl3_ref, bl3b_ref, bl4_ref, bns_ref,
                mu_ref, hw_ref, beta_ref):
    xs = xs_ref[...]

    mu = jnp.tanh(_dotd(xs, f11_ref[...], (((1,), (1,)))) + f11b_ref[...])
    mu = jnp.tanh(_dotd(mu, f12_ref[...], (((1,), (1,)))) + f12b_ref[...])
    mu_ref[...] = jnp.tanh(
        jnp.sum(_bf(mu).astype(jnp.float32)
                * _bf(f13_ref[...]).astype(jnp.float32),
                axis=1, keepdims=True) + bns_ref[0, 6])

    hw_ref[...] = _dotd(xs, fc2_ref[...], (((1,), (1,))))

    def lnorm(y, g_s, b_s):
        m = jnp.mean(y)
        v = jnp.mean((y - m) * (y - m))
        return (y - m) / jnp.sqrt(v + 1e-5) * g_s + b_s

    z = jnp.tanh(lnorm(_dotd(xs, bl1_ref[...], (((1,), (1,)))) + bl1b_ref[...],
                       bns_ref[0, 0], bns_ref[0, 1]))
    z = jnp.tanh(lnorm(_dotd(z, bl2_ref[...], (((1,), (1,)))) + bl2b_ref[...],
                       bns_ref[0, 2], bns_ref[0, 3]))
    z = jnp.tanh(lnorm(_dotd(z, bl3_ref[...], (((1,), (1,)))) + bl3b_ref[...],
                       bns_ref[0, 4], bns_ref[0, 5]))
    beta_ref[...] = (jnp.sum(_bf(z).astype(jnp.float32)
                             * _bf(bl4_ref[...]).astype(jnp.float32),
                             axis=1, keepdims=True) + bns_ref[0, 7])


def _heads(xs, p):
    bns = jnp.stack([p['b1_g'][0], p['b1_b'][0], p['b2_g'][0], p['b2_b'][0],
                     p['b3_g'][0], p['b3_b'][0], p['fc13_b'][0],
                     p['bl4_b'][0]]).reshape(1, 8)
    args = (xs,
            p['fc11_W'], p['fc11_b'].reshape(1, -1),
            p['fc12_W'], p['fc12_b'].reshape(1, -1),
            p['fc13_W'],
            p['fc2_W'],
            p['bl1_W'], p['bl1_b'].reshape(1, -1),
            p['bl2_W'], p['bl2_b'].reshape(1, -1),
            p['bl3_W'], p['bl3_b'].reshape(1, -1),
            p['bl4_W'])
    in_specs = [pl.BlockSpec(a.shape, lambda: (0, 0)) for a in args]
    in_specs.append(pl.BlockSpec(memory_space=pltpu.SMEM))
    return pl.pallas_call(
        _heads_body,
        in_specs=in_specs,
        out_specs=[pl.BlockSpec((N, 1), lambda: (0, 0)),
                   pl.BlockSpec((N, F0), lambda: (0, 0)),
                   pl.BlockSpec((N, 1), lambda: (0, 0))],
        out_shape=[jax.ShapeDtypeStruct((N, 1), jnp.float32),
                   jax.ShapeDtypeStruct((N, F0), jnp.float32),
                   jax.ShapeDtypeStruct((N, 1), jnp.float32)],
    )(*args, bns)


def _temp_body(hwb_ref, hw_ref, out_ref):
    out_ref[...] = jnp.tanh(_dotd(hwb_ref[...], hw_ref[...], (((1,), (1,)))))


def _temp(hw, rblk):
    return pl.pallas_call(
        _temp_body,
        grid=(N // rblk,),
        in_specs=[pl.BlockSpec((rblk, F0), lambda i: (i, 0)),
                  pl.BlockSpec((N, F0), lambda i: (0, 0))],
        out_specs=pl.BlockSpec((rblk, N), lambda i: (i, 0)),
        out_shape=jax.ShapeDtypeStruct((N, N), jnp.float32),
    )(hw, hw)


def _knn_body(db_ref, mu_ref, wm_ref, w_ref, *, rblk):
    dist = db_ref[...]
    lane = jax.lax.broadcasted_iota(jnp.int32, (rblk, N), 1)
    big = jnp.float32(jnp.inf)
    acc = jnp.zeros((rblk, N), jnp.int32)
    for k in range(K_NN + 1):
        vmin = jnp.min(dist, axis=1, keepdims=True)
        cand = jnp.where(dist == vmin, lane, N)
        imin = jnp.min(cand, axis=1, keepdims=True)
        onehot = lane == imin
        if k > 0:
            acc = acc + onehot.astype(jnp.int32)
        dist = jnp.where(onehot, big, dist)
    wm_ref[...] = acc
    w_ref[...] = jnp.sum(acc.astype(jnp.float32)
                         * _bf(mu_ref[...]).astype(jnp.float32),
                         axis=1, keepdims=True)


def _knn(D, mu_lane, rblk):
    body = functools.partial(_knn_body, rblk=rblk)
    return pl.pallas_call(
        body,
        grid=(N // rblk,),
        in_specs=[pl.BlockSpec((rblk, N), lambda i: (i, 0)),
                  pl.BlockSpec((1, N), lambda i: (0, 0))],
        out_specs=[pl.BlockSpec((rblk, N), lambda i: (i, 0)),
                   pl.BlockSpec((rblk, 1), lambda i: (i, 0))],
        out_shape=[jax.ShapeDtypeStruct((N, N), jnp.int32),
                   jax.ShapeDtypeStruct((N, 1), jnp.float32)],
    )(D, mu_lane)


def _final_body(w_ref, r_ref, beta_ref, wn_ref, R_ref, pre_ref):
    w = w_ref[...]

    def smax(a):
        amax = jnp.max(a)
        e = jnp.exp(a - amax)
        return e / jnp.sum(e)

    wn = smax(-50.0 * jnp.exp(-8.0 * w)) - smax(-50.0 * jnp.exp(8.0 * w))
    wn_ref[...] = wn
    Rv = jnp.sum(_bf(r_ref[...]).astype(jnp.float32)
                 * _bf(wn).astype(jnp.float32), axis=1, keepdims=True)
    R_ref[...] = Rv
    pre_ref[...] = (_bf(beta_ref[...]).astype(jnp.float32)
                    * _bf(Rv).astype(jnp.float32))


def _final(w_lane, r2, beta_lane):
    return pl.pallas_call(
        _final_body,
        in_specs=[pl.BlockSpec((1, N), lambda: (0, 0)),
                  pl.BlockSpec((T, N), lambda: (0, 0)),
                  pl.BlockSpec((1, N), lambda: (0, 0))],
        out_specs=[pl.BlockSpec((1, N), lambda: (0, 0)),
                   pl.BlockSpec((T, 1), lambda: (0, 0)),
                   pl.BlockSpec((T, N), lambda: (0, 0))],
        out_shape=[jax.ShapeDtypeStruct((1, N), jnp.float32),
                   jax.ShapeDtypeStruct((T, 1), jnp.float32),
                   jax.ShapeDtypeStruct((T, N), jnp.float32)],
    )(w_lane, r2, beta_lane)


def _bn(xx, g, b):
    m = xx.mean(axis=(0, 2, 3), keepdims=True)
    v = xx.var(axis=(0, 2, 3), keepdims=True)
    return (xx - m) / jnp.sqrt(v + 1e-5) * g[None, :, None, None] + b[None, :, None, None]


def _gcn(xx, s, d, norm, theta):
    h = xx @ theta.T
    outs = []
    for t in range(h.shape[1]):
        ht = h[:, t]
        msg = jnp.take(ht, s, axis=1) * norm[None, :, None]
        outs.append(jnp.zeros_like(ht).at[:, d, :].add(msg))
    return jnp.stack(outs, axis=1)


def _antisym(xx, s, d, norm, W, b, theta):
    aw = W - W.T - 0.1 * jnp.eye(W.shape[0], dtype=xx.dtype)
    h = _gcn(xx, s, d, norm, theta)
    h = xx @ aw.T + h + b
    return xx + 0.1 * jnp.tanh(h)


def kernel(x, edge_index, r, h, c, params):
    p = params
    src, dst = edge_index[0], edge_index[1]
    loop = jnp.arange(N, dtype=src.dtype)
    s = jnp.concatenate([src, loop])
    d = jnp.concatenate([dst, loop])
    deg = jnp.zeros((N,), jnp.float32).at[d].add(1.0)
    dinv = jnp.where(deg > 0, 1.0 / jnp.sqrt(deg), 0.0)
    norm = dinv[s] * dinv[d]

    xt = jnp.transpose(x, (0, 2, 1, 3))
    xt = _bn(xt, p['bn1_g'], p['bn1_b'])
    x1 = jnp.transpose(xt, (0, 2, 1, 3))
    x1 = jnp.tanh(_antisym(x1, s, d, norm, p['as1_W'], p['as1_b'], p['as1_t']))
    x1 = jnp.tanh(_antisym(x1, s, d, norm, p['as2_W'], p['as2_b'], p['as2_t']))
    x1 = jnp.tanh(x1 @ p['fc0_W'].T + p['fc0_b'])
    x1 = jnp.transpose(x1, (0, 3, 2, 1))
    x1 = x1 @ p['l40_W'].T + p['l40_b']
    x1 = jnp.tanh(jnp.transpose(x1, (0, 3, 2, 1)))
    xs = x1[:, 0]
    ht = h[0]
    ct = c[0]
    for t in range(xs.shape[0]):
        g = xs[t] @ p['lstm_Wih'].T + p['lstm_bih'] + ht @ p['lstm_Whh'].T + p['lstm_bhh']
        gi, gf, gg, go = jnp.split(g, 4, axis=-1)
        ct = jax.nn.sigmoid(gf) * ct + jax.nn.sigmoid(gi) * jnp.tanh(gg)
        ht = jax.nn.sigmoid(go) * jnp.tanh(ct)
    H = ht
    sq = jnp.sum(H * H, axis=1)
    d2 = sq[:, None] + sq[None, :] - 2.0 * (H @ H.T)
    D = jnp.sqrt(jnp.maximum(d2, 0.0))

    xs2 = xs[0]
    mu2, hw2, beta2 = _heads(xs2, p)
    temp2 = _temp(hw2, N // 4)
    Wm2, w2 = _knn(D, mu2.reshape(1, N), N // 4)
    wn2, R2, pre2 = _final(w2.reshape(1, N), r[0], beta2.reshape(1, N))

    return (R2.reshape(1, T), wn2.reshape(1, N, 1), xs, mu2[None],
            temp2[None], Wm2[None], pre2[None])
